# baseline (speedup 1.0000x reference)
import functools

import jax
import jax.numpy as jnp
from jax import lax
from jax.experimental import pallas as pl
from jax.experimental.pallas import tpu as pltpu
from jax.experimental.pallas import tpu_sc as plsc

NUM_CORES = 2
NUM_SUBCORES = 16
NW = NUM_CORES * NUM_SUBCORES

CHUNK = 1024


def _make_sc_gather(npix: int, nfreq: int, nrows: int):
    assert nrows % NW == 0
    rows_per_w = nrows // NW
    assert rows_per_w % CHUNK == 0
    n_chunks = rows_per_w // CHUNK
    mesh = plsc.VectorSubcoreMesh(core_axis_name="c", subcore_axis_name="s")

    @functools.partial(
        pl.kernel,
        mesh=mesh,
        compiler_params=pltpu.CompilerParams(use_tc_tiling_on_sc=False),
        out_type=jax.ShapeDtypeStruct((nrows, nfreq), jnp.float32),
        scratch_types=[
            pltpu.VMEM((CHUNK,), jnp.int32),
            pltpu.VMEM((CHUNK, nfreq), jnp.float32),
            pltpu.SemaphoreType.DMA,
        ],
    )
    def sc_gather(table_hbm, idx_hbm, out_hbm, idx_v, rows_v, sem):
        wid = lax.axis_index("s") * NUM_CORES + lax.axis_index("c")
        base_w = wid * rows_per_w

        def body(c, carry):
            base = base_w + c * CHUNK
            pltpu.sync_copy(idx_hbm.at[pl.ds(base, CHUNK)], idx_v)
            pltpu.async_copy(table_hbm.at[idx_v], rows_v, sem).wait()
            pltpu.sync_copy(rows_v, out_hbm.at[pl.ds(base, CHUNK)])
            return carry

        lax.fori_loop(0, n_chunks, body, 0)

    return sc_gather


def _combine_body(g_ref, w_ref, o_ref):
    g = g_ref[...]
    w = w_ref[...]
    acc = jnp.sum(g * w[:, :, None], axis=0)
    o_ref[...] = jnp.abs(acc.T)


def kernel(params, inds, wgts):
    npol, npol2, nmodel, nfreq, npix = params.shape
    nnbr, nsrc = inds.shape

    table = jnp.transpose(params.reshape(nfreq, npix))
    idx_flat = inds.reshape(-1)

    gathered = _make_sc_gather(npix, nfreq, nnbr * nsrc)(table, idx_flat)

    BS = 2048
    out_t = pl.pallas_call(
        _combine_body,
        grid=(nsrc // BS,),
        in_specs=[
            pl.BlockSpec((nnbr, BS, nfreq), lambda i: (0, i, 0)),
            pl.BlockSpec((nnbr, BS), lambda i: (0, i)),
        ],
        out_specs=pl.BlockSpec((nfreq, BS), lambda i: (0, i)),
        out_shape=jax.ShapeDtypeStruct((nfreq, nsrc), jnp.float32),
    )(gathered.reshape(nnbr, nsrc, nfreq), wgts)

    return out_t.reshape(npol, npol2, nmodel, nfreq, nsrc)
